# initial kernel scaffold (unmeasured)
import jax
import jax.numpy as jnp
from jax import lax
from jax.experimental import pallas as pl
from jax.experimental.pallas import tpu as pltpu

B = 4
S = 1024
S_HALF = S // 2
HD = 2048
N = 4096


def kernel(O, Wo):
    O3 = O.reshape(B, S, HD).astype(jnp.bfloat16)
    Wo_b = Wo.astype(jnp.bfloat16)

    def body(o_ref, w_ref, out_ref, send_buf, recv_buf, send_sems, recv_sems):
        my_x = lax.axis_index("x")
        my_y = lax.axis_index("y")
        my_z = lax.axis_index("z")
        other_x = 1 - my_x
        partner = (other_x, my_y, my_z)

        my_start = my_x * S_HALF
        other_start = other_x * S_HALF

        barrier_sem = pltpu.get_barrier_semaphore()
        pl.semaphore_signal(
            barrier_sem, inc=1, device_id=partner,
            device_id_type=pl.DeviceIdType.MESH,
        )
        pl.semaphore_wait(barrier_sem, 1)

        rdmas = []
        for b in range(B):
            acc = jnp.dot(
                o_ref[b, pl.ds(other_start, S_HALF), :], w_ref[...],
                preferred_element_type=jnp.float32,
            )
            send_buf[b] = acc.astype(jnp.bfloat16)
            rdma = pltpu.make_async_remote_copy(
                src_ref=send_buf.at[b],
                dst_ref=recv_buf.at[b],
                send_sem=send_sems.at[b],
                recv_sem=recv_sems.at[b],
                device_id=partner,
                device_id_type=pl.DeviceIdType.MESH,
            )
            rdma.start()
            rdmas.append(rdma)

        for b in range(B):
            out_ref[b] = jnp.dot(
                o_ref[b, pl.ds(my_start, S_HALF), :], w_ref[...],
                preferred_element_type=jnp.float32,
            )

        for b in range(B):
            rdmas[b].wait_recv()
            out_ref[b] += recv_buf[b].astype(jnp.float32)

        for b in range(B):
            rdmas[b].wait_send()

    return pl.pallas_call(
        body,
        out_shape=jax.ShapeDtypeStruct((B, S_HALF, N), jnp.float32),
        in_specs=[
            pl.BlockSpec(memory_space=pltpu.VMEM),
            pl.BlockSpec(memory_space=pltpu.VMEM),
        ],
        out_specs=pl.BlockSpec(memory_space=pltpu.VMEM),
        scratch_shapes=[
            pltpu.VMEM((B, S_HALF, N), jnp.bfloat16),
            pltpu.VMEM((B, S_HALF, N), jnp.bfloat16),
            pltpu.SemaphoreType.DMA((B,)),
            pltpu.SemaphoreType.DMA((B,)),
        ],
        compiler_params=pltpu.CompilerParams(collective_id=0),
    )(O3, Wo_b)


# baseline (device time: 267678 ns/iter reference)
import jax
import jax.numpy as jnp
from jax import lax
from jax.experimental import pallas as pl
from jax.experimental.pallas import tpu as pltpu

B = 4
S = 1024
S_HALF = S // 2
HD = 2048
N = 4096
ROWS = 256
N_CHUNK = B * S_HALF // ROWS
PER_B = S_HALF // ROWS


def kernel(O, Wo):
    O3 = O.reshape(B, S, HD).astype(jnp.bfloat16)
    Wo_b = Wo.astype(jnp.bfloat16)

    def body(o_hbm, w_ref, out_hbm, o_buf, send_buf, recv_buf, staging,
             o_sems, send_sems, recv_sems, out_sems):
        my_x = lax.axis_index("x")
        my_y = lax.axis_index("y")
        my_z = lax.axis_index("z")
        other_x = 1 - my_x
        partner = (other_x, my_y, my_z)

        my_start = my_x * S_HALF
        other_start = other_x * S_HALF

        barrier_sem = pltpu.get_barrier_semaphore()
        pl.semaphore_signal(
            barrier_sem, inc=1, device_id=partner,
            device_id_type=pl.DeviceIdType.MESH,
        )
        pl.semaphore_wait(barrier_sem, 1)

        def o_chunk_dma(k):
            c = k % N_CHUNK
            b = c // PER_B
            off = (c % PER_B) * ROWS
            half = other_start if k < N_CHUNK else my_start
            return pltpu.make_async_copy(
                o_hbm.at[b, pl.ds(half + off, ROWS), :],
                o_buf.at[k % 2],
                o_sems.at[k % 2],
            )

        rdmas = []
        out_copies = []
        o_chunk_dma(0).start()
        for k in range(2 * N_CHUNK):
            o_chunk_dma(k).wait()
            if k + 1 < 2 * N_CHUNK:
                o_chunk_dma(k + 1).start()

            if k < N_CHUNK:
                c = k
                if c >= 2:
                    rdmas[c - 2].wait_send()
                send_buf[c % 2] = jnp.dot(
                    o_buf[k % 2], w_ref[...],
                    preferred_element_type=jnp.float32,
                ).astype(jnp.bfloat16)
                rdma = pltpu.make_async_remote_copy(
                    src_ref=send_buf.at[c % 2],
                    dst_ref=recv_buf.at[c],
                    send_sem=send_sems.at[c],
                    recv_sem=recv_sems.at[c],
                    device_id=partner,
                    device_id_type=pl.DeviceIdType.MESH,
                )
                rdma.start()
                rdmas.append(rdma)
            else:
                c = k - N_CHUNK
                b = c // PER_B
                off = (c % PER_B) * ROWS
                if c >= 2:
                    out_copies[c - 2].wait()
                staging[c % 2] = jnp.dot(
                    o_buf[k % 2], w_ref[...],
                    preferred_element_type=jnp.float32,
                )
                rdmas[c].wait_recv()
                staging[c % 2] += recv_buf[c].astype(jnp.float32)
                out_copy = pltpu.make_async_copy(
                    staging.at[c % 2],
                    out_hbm.at[b, pl.ds(off, ROWS), :],
                    out_sems.at[c % 2],
                )
                out_copy.start()
                out_copies.append(out_copy)

        for c in range(N_CHUNK - 2, N_CHUNK):
            rdmas[c].wait_send()
            out_copies[c].wait()

    return pl.pallas_call(
        body,
        out_shape=jax.ShapeDtypeStruct((B, S_HALF, N), jnp.float32),
        in_specs=[
            pl.BlockSpec(memory_space=pl.ANY),
            pl.BlockSpec(memory_space=pltpu.VMEM),
        ],
        out_specs=pl.BlockSpec(memory_space=pl.ANY),
        scratch_shapes=[
            pltpu.VMEM((2, ROWS, HD), jnp.bfloat16),
            pltpu.VMEM((2, ROWS, N), jnp.bfloat16),
            pltpu.VMEM((N_CHUNK, ROWS, N), jnp.bfloat16),
            pltpu.VMEM((2, ROWS, N), jnp.float32),
            pltpu.SemaphoreType.DMA((2,)),
            pltpu.SemaphoreType.DMA((N_CHUNK,)),
            pltpu.SemaphoreType.DMA((N_CHUNK,)),
            pltpu.SemaphoreType.DMA((2,)),
        ],
        compiler_params=pltpu.CompilerParams(
            collective_id=0,
            vmem_limit_bytes=60 * 1024 * 1024,
        ),
    )(O3, Wo_b)


# device time: 242893 ns/iter; 1.1020x vs baseline; 1.1020x over previous
import jax
import jax.numpy as jnp
from jax import lax
from jax.experimental import pallas as pl
from jax.experimental.pallas import tpu as pltpu

B = 4
S = 1024
S_HALF = S // 2
H = 16
D = 128
HD = H * D
N = 4096
ROWS = 256
N_CHUNK = B * S_HALF // ROWS
PER_B = S_HALF // ROWS


def kernel(O, Wo):
    Wo_b = Wo.astype(jnp.bfloat16)

    def body(o_hbm, w_ref, out_hbm, o_buf, send_buf, recv_buf, staging,
             o_sems, send_sems, recv_sems, out_sems):
        my_x = lax.axis_index("x")
        my_y = lax.axis_index("y")
        my_z = lax.axis_index("z")
        other_x = 1 - my_x
        partner = (other_x, my_y, my_z)

        my_start = my_x * S_HALF
        other_start = other_x * S_HALF

        barrier_sem = pltpu.get_barrier_semaphore()
        pl.semaphore_signal(
            barrier_sem, inc=1, device_id=partner,
            device_id_type=pl.DeviceIdType.MESH,
        )
        pl.semaphore_wait(barrier_sem, 1)

        def o_chunk_dma(k):
            c = k % N_CHUNK
            b = c // PER_B
            off = (c % PER_B) * ROWS
            half = other_start if k < N_CHUNK else my_start
            return pltpu.make_async_copy(
                o_hbm.reshape(B, S, HD).at[b, pl.ds(half + off, ROWS), :],
                o_buf.at[k % 2],
                o_sems.at[k % 2],
            )

        rdmas = []
        out_copies = []
        o_chunk_dma(0).start()
        for k in range(2 * N_CHUNK):
            o_chunk_dma(k).wait()
            if k + 1 < 2 * N_CHUNK:
                o_chunk_dma(k + 1).start()

            if k < N_CHUNK:
                c = k
                if c >= 2:
                    rdmas[c - 2].wait_send()
                send_buf[c % 2] = jnp.dot(
                    o_buf[k % 2].astype(jnp.bfloat16), w_ref[...],
                    preferred_element_type=jnp.float32,
                ).astype(jnp.bfloat16)
                rdma = pltpu.make_async_remote_copy(
                    src_ref=send_buf.at[c % 2],
                    dst_ref=recv_buf.at[c],
                    send_sem=send_sems.at[c],
                    recv_sem=recv_sems.at[c],
                    device_id=partner,
                    device_id_type=pl.DeviceIdType.MESH,
                )
                rdma.start()
                rdmas.append(rdma)
            else:
                c = k - N_CHUNK
                b = c // PER_B
                off = (c % PER_B) * ROWS
                if c >= 2:
                    out_copies[c - 2].wait()
                local = jnp.dot(
                    o_buf[k % 2].astype(jnp.bfloat16), w_ref[...],
                    preferred_element_type=jnp.float32,
                )
                rdmas[c].wait_recv()
                staging[c % 2] = (
                    local + recv_buf[c].astype(jnp.float32)
                ).astype(jnp.bfloat16)
                out_copy = pltpu.make_async_copy(
                    staging.at[c % 2],
                    out_hbm.at[b, pl.ds(off, ROWS), :],
                    out_sems.at[c % 2],
                )
                out_copy.start()
                out_copies.append(out_copy)

        for c in range(N_CHUNK - 2, N_CHUNK):
            rdmas[c].wait_send()
            out_copies[c].wait()

    out_bf16 = pl.pallas_call(
        body,
        out_shape=jax.ShapeDtypeStruct((B, S_HALF, N), jnp.bfloat16),
        in_specs=[
            pl.BlockSpec(memory_space=pl.ANY),
            pl.BlockSpec(memory_space=pltpu.VMEM),
        ],
        out_specs=pl.BlockSpec(memory_space=pl.ANY),
        scratch_shapes=[
            pltpu.VMEM((2, ROWS, HD), jnp.float32),
            pltpu.VMEM((2, ROWS, N), jnp.bfloat16),
            pltpu.VMEM((N_CHUNK, ROWS, N), jnp.bfloat16),
            pltpu.VMEM((2, ROWS, N), jnp.bfloat16),
            pltpu.SemaphoreType.DMA((2,)),
            pltpu.SemaphoreType.DMA((N_CHUNK,)),
            pltpu.SemaphoreType.DMA((N_CHUNK,)),
            pltpu.SemaphoreType.DMA((2,)),
        ],
        compiler_params=pltpu.CompilerParams(
            collective_id=0,
            vmem_limit_bytes=63 * 1024 * 1024,
        ),
    )(O, Wo_b)
    return out_bf16.astype(jnp.float32)
